# Initial kernel scaffold; baseline (speedup 1.0000x reference)
import jax
import jax.numpy as jnp
from jax.experimental import pallas as pl


def kernel(x, alpha, edge_index, W1s, W1n, b1, g1, be1, W2s, W2n, b2, Wm1, bm1, gm1, bem1, Wm2, bm2, temp, Wc, bc):
    raise NotImplementedError("write your pallas kernel here")


# speedup vs baseline: 3.9142x; 3.9142x over previous
import functools

import jax
import jax.numpy as jnp
from jax import lax
from jax.experimental import pallas as pl
from jax.experimental.pallas import tpu as pltpu
from jax.experimental.pallas import tpu_sc as plsc

N = 10000
D = 128
E = 320000
K = 100
NS = 16
NR = 10240
ROWS_PT = NR // NS
CHUNKS = E // K
CH_PT = CHUNKS // NS
IB = 8
NB = CH_PT // IB

_MESH = plsc.VectorSubcoreMesh(core_axis_name="c", subcore_axis_name="s",
                               num_cores=1)


def _seg_body(with_deg, table_hbm, srcr_hbm, tgtr_hbm, ones_hbm, zrow_hbm,
              zdeg_hbm, agg_hbm, deg_hbm, sidx, tidx, rows, onev, acc, dacc,
              sem):
    s = lax.axis_index("s")
    pltpu.sync_copy(zrow_hbm.at[pl.ds(s * ROWS_PT, ROWS_PT)],
                    acc.at[pl.ds(s * ROWS_PT, ROWS_PT)])
    if with_deg:
        pltpu.sync_copy(ones_hbm, onev)
        pltpu.sync_copy(zdeg_hbm.at[pl.ds(s * ROWS_PT, ROWS_PT)],
                        dacc.at[pl.ds(s * ROWS_PT, ROWS_PT)])

    plsc.subcore_barrier()
    my_src = srcr_hbm.at[s]
    my_tgt = tgtr_hbm.at[s]

    def outer(b, carry):
        pltpu.sync_copy(my_src.at[pl.ds(b * IB, IB)], sidx)
        pltpu.sync_copy(my_tgt.at[pl.ds(b * IB, IB)], tidx)

        def inner(j, carry2):
            pltpu.async_copy(table_hbm.at[sidx.at[j]], rows, sem).wait()
            pltpu.sync_copy(rows, acc.at[tidx.at[j]], add=True)
            if with_deg:
                pltpu.sync_copy(onev, dacc.at[tidx.at[j]], add=True)
            return carry2

        lax.fori_loop(0, IB, inner, 0)
        return carry

    lax.fori_loop(0, NB, outer, 0)
    plsc.subcore_barrier()
    pltpu.sync_copy(acc.at[pl.ds(s * ROWS_PT, ROWS_PT)],
                    agg_hbm.at[pl.ds(s * ROWS_PT, ROWS_PT)])
    if with_deg:
        pltpu.sync_copy(dacc.at[pl.ds(s * ROWS_PT, ROWS_PT)],
                        deg_hbm.at[pl.ds(s * ROWS_PT, ROWS_PT)])


_seg_deg = functools.partial(
    pl.kernel,
    out_type=(jax.ShapeDtypeStruct((NR, D), jnp.float32),
              jax.ShapeDtypeStruct((NR,), jnp.float32)),
    mesh=_MESH,
    scratch_types=[
        pltpu.VMEM((IB, K), jnp.int32),
        pltpu.VMEM((IB, K), jnp.int32),
        pltpu.VMEM((K, D), jnp.float32),
        pltpu.VMEM((K,), jnp.float32),
        pltpu.VMEM_SHARED((NR, D), jnp.float32),
        pltpu.VMEM_SHARED((NR,), jnp.float32),
        pltpu.SemaphoreType.DMA,
    ],
)(functools.partial(_seg_body, True))


def _seg_nodeg_body(table_hbm, srcr_hbm, tgtr_hbm, zrow_hbm, agg_hbm,
                    sidx, tidx, rows, acc, sem):
    _seg_body(False, table_hbm, srcr_hbm, tgtr_hbm, None, zrow_hbm, None,
              agg_hbm, None, sidx, tidx, rows, None, acc, None, sem)


_seg = functools.partial(
    pl.kernel,
    out_type=jax.ShapeDtypeStruct((NR, D), jnp.float32),
    mesh=_MESH,
    scratch_types=[
        pltpu.VMEM((IB, K), jnp.int32),
        pltpu.VMEM((IB, K), jnp.int32),
        pltpu.VMEM((K, D), jnp.float32),
        pltpu.VMEM_SHARED((NR, D), jnp.float32),
        pltpu.SemaphoreType.DMA,
    ],
)(_seg_nodeg_body)


def _bn(z, g, be):
    mu = jnp.mean(z, axis=0, keepdims=True)
    zc = z - mu
    var = jnp.mean(zc * zc, axis=0, keepdims=True)
    return g * zc * lax.rsqrt(var + 1e-5) + be


def _dense_a(al_ref, wm1_ref, bm1_ref, gm1_ref, bem1_ref, wm2_ref,
             bm2_ref, an_ref, hp_ref, ss_ref):
    al = al_ref[...]
    nrm = jnp.sqrt(jnp.sum(al * al, axis=1, keepdims=True))
    an = al / jnp.maximum(nrm, 1e-12)
    an_ref[...] = an
    ss_ref[...] = jnp.sum(an * an, axis=1, keepdims=True)
    z = jnp.dot(al, wm1_ref[...], preferred_element_type=jnp.float32) + bm1_ref[...]
    m = jax.nn.sigmoid(_bn(z, gm1_ref[...], bem1_ref[...]))
    hp_ref[...] = (jnp.dot(m, wm2_ref[...], preferred_element_type=jnp.float32)
                   + bm2_ref[...])


def _dense_b(x_ref, aggx_ref, deg_ref, w1s_ref, w1n_ref, b1_ref, g1_ref,
             be1_ref, h_ref):
    inv = 1.0 / jnp.maximum(deg_ref[...], 1.0)
    mean1 = aggx_ref[:N] * inv
    z = (jnp.dot(x_ref[...], w1s_ref[...], preferred_element_type=jnp.float32)
         + jnp.dot(mean1, w1n_ref[...], preferred_element_type=jnp.float32)
         + b1_ref[...])
    h_ref[...] = jnp.maximum(_bn(z, g1_ref[...], be1_ref[...]), 0.0)


def _dense_c(h_ref, aggh_ref, deg_ref, an_ref, aggan_ref, ss_ref, hp_ref,
             w2s_ref, w2n_ref, b2_ref, temp_ref, wc_ref, bc_ref, out_ref):
    deg = deg_ref[...]
    inv = 1.0 / jnp.maximum(deg, 1.0)
    mean2 = aggh_ref[:N] * inv
    hg = (jnp.dot(h_ref[...], w2s_ref[...], preferred_element_type=jnp.float32)
          + jnp.dot(mean2, w2n_ref[...], preferred_element_type=jnp.float32)
          + b2_ref[...])
    ssum = jnp.sum(an_ref[...] * aggan_ref[:N], axis=1, keepdims=True) + ss_ref[...]
    gate = jax.nn.sigmoid(temp_ref[...] * ssum / (deg + 1.0))
    z = jnp.maximum(gate * hg + (1.0 - gate) * hp_ref[...], 0.0)
    logits = (jnp.dot(z, wc_ref[...], preferred_element_type=jnp.float32)
              + bc_ref[...])
    mx = jnp.max(logits, axis=1, keepdims=True)
    e = jnp.exp(logits - mx)
    out_ref[...] = e / jnp.sum(e, axis=1, keepdims=True)


def kernel(x, alpha, edge_index, W1s, W1n, b1, g1, be1, W2s, W2n, b2,
           Wm1, bm1, gm1, bem1, Wm2, bm2, temp, Wc, bc):
    f32 = jnp.float32
    src = edge_index[0].reshape(NS, CH_PT, K)
    tgt = edge_index[1].reshape(NS, CH_PT, K)
    ones_k = jnp.ones((K,), f32)
    zrow = jnp.zeros((NR, D), f32)
    zdeg = jnp.zeros((NR,), f32)

    an, hp, ss = pl.pallas_call(
        _dense_a,
        out_shape=(jax.ShapeDtypeStruct((N, D), f32),
                   jax.ShapeDtypeStruct((N, D), f32),
                   jax.ShapeDtypeStruct((N, 1), f32)),
    )(alpha, Wm1, bm1.reshape(1, -1), gm1.reshape(1, -1),
      bem1.reshape(1, -1), Wm2, bm2.reshape(1, -1))

    aggx, degp = _seg_deg(x, src, tgt, ones_k, zrow, zdeg)
    aggan = _seg(an, src, tgt, zrow)
    deg = degp[:N].reshape(N, 1)

    h = pl.pallas_call(
        _dense_b,
        out_shape=jax.ShapeDtypeStruct((N, D), f32),
    )(x, aggx, deg, W1s, W1n, b1.reshape(1, -1), g1.reshape(1, -1),
      be1.reshape(1, -1))

    aggh = _seg(h, src, tgt, zrow)

    out = pl.pallas_call(
        _dense_c,
        out_shape=jax.ShapeDtypeStruct((N, 40), f32),
    )(h, aggh, deg, an, aggan, ss, hp, W2s, W2n, b2.reshape(1, -1),
      temp.reshape(1, 1), Wc, bc.reshape(1, -1))
    return out


# speedup vs baseline: 6.5723x; 1.6791x over previous
import functools

import jax
import jax.numpy as jnp
from jax import lax
from jax.experimental import pallas as pl
from jax.experimental.pallas import tpu as pltpu
from jax.experimental.pallas import tpu_sc as plsc

N = 10000
D = 128
E = 320000
K = 100
NC = 2
NS = 16
NW = NC * NS
NR = 10240
ROWS_PT = NR // NS
CHUNKS = E // K
CH_PW = CHUNKS // NW
IB = 4
NB = CH_PW // IB

_MESH = plsc.VectorSubcoreMesh(core_axis_name="c", subcore_axis_name="s")


def _seg_body(with_deg, table_hbm, srcr_hbm, tgtr_hbm, ones_hbm, zrow_hbm,
              zdeg_hbm, agg_hbm, deg_hbm, sidx, tidx, rows, onev, acc, dacc,
              sem):
    c = lax.axis_index("c")
    s = lax.axis_index("s")
    w = s * NC + c
    pltpu.sync_copy(zrow_hbm.at[pl.ds(s * ROWS_PT, ROWS_PT)],
                    acc.at[pl.ds(s * ROWS_PT, ROWS_PT)])
    if with_deg:
        pltpu.sync_copy(ones_hbm, onev)
        pltpu.sync_copy(zdeg_hbm.at[pl.ds(s * ROWS_PT, ROWS_PT)],
                        dacc.at[pl.ds(s * ROWS_PT, ROWS_PT)])

    plsc.subcore_barrier()
    my_src = srcr_hbm.at[w]
    my_tgt = tgtr_hbm.at[w]

    def outer(b, carry):
        pltpu.sync_copy(my_src.at[pl.ds(b * IB, IB)], sidx)
        pltpu.sync_copy(my_tgt.at[pl.ds(b * IB, IB)], tidx)

        def inner(j, carry2):
            pltpu.async_copy(table_hbm.at[sidx.at[j]], rows, sem).wait()
            pltpu.sync_copy(rows, acc.at[tidx.at[j]], add=True)
            if with_deg:
                pltpu.sync_copy(onev, dacc.at[tidx.at[j]], add=True)
            return carry2

        lax.fori_loop(0, IB, inner, 0)
        return carry

    lax.fori_loop(0, NB, outer, 0)
    plsc.subcore_barrier()
    pltpu.sync_copy(acc.at[pl.ds(s * ROWS_PT, ROWS_PT)],
                    agg_hbm.at[c].at[pl.ds(s * ROWS_PT, ROWS_PT)])
    if with_deg:
        pltpu.sync_copy(dacc.at[pl.ds(s * ROWS_PT, ROWS_PT)],
                        deg_hbm.at[c].at[pl.ds(s * ROWS_PT, ROWS_PT)])


_seg_deg = functools.partial(
    pl.kernel,
    out_type=(jax.ShapeDtypeStruct((NC, NR, D), jnp.float32),
              jax.ShapeDtypeStruct((NC, NR), jnp.float32)),
    mesh=_MESH,
    scratch_types=[
        pltpu.VMEM((IB, K), jnp.int32),
        pltpu.VMEM((IB, K), jnp.int32),
        pltpu.VMEM((K, D), jnp.float32),
        pltpu.VMEM((K,), jnp.float32),
        pltpu.VMEM_SHARED((NR, D), jnp.float32),
        pltpu.VMEM_SHARED((NR,), jnp.float32),
        pltpu.SemaphoreType.DMA,
    ],
)(functools.partial(_seg_body, True))


def _seg_nodeg_body(table_hbm, srcr_hbm, tgtr_hbm, zrow_hbm, agg_hbm,
                    sidx, tidx, rows, acc, sem):
    _seg_body(False, table_hbm, srcr_hbm, tgtr_hbm, None, zrow_hbm, None,
              agg_hbm, None, sidx, tidx, rows, None, acc, None, sem)


_seg = functools.partial(
    pl.kernel,
    out_type=jax.ShapeDtypeStruct((NC, NR, D), jnp.float32),
    mesh=_MESH,
    scratch_types=[
        pltpu.VMEM((IB, K), jnp.int32),
        pltpu.VMEM((IB, K), jnp.int32),
        pltpu.VMEM((K, D), jnp.float32),
        pltpu.VMEM_SHARED((NR, D), jnp.float32),
        pltpu.SemaphoreType.DMA,
    ],
)(_seg_nodeg_body)


def _bn(z, g, be):
    mu = jnp.mean(z, axis=0, keepdims=True)
    zc = z - mu
    var = jnp.mean(zc * zc, axis=0, keepdims=True)
    return g * zc * lax.rsqrt(var + 1e-5) + be


def _dense_a(al_ref, wm1_ref, bm1_ref, gm1_ref, bem1_ref, wm2_ref,
             bm2_ref, an_ref, hp_ref, ss_ref):
    al = al_ref[...]
    nrm = jnp.sqrt(jnp.sum(al * al, axis=1, keepdims=True))
    an = al / jnp.maximum(nrm, 1e-12)
    an_ref[...] = an
    ss_ref[...] = jnp.sum(an * an, axis=1, keepdims=True)
    z = jnp.dot(al, wm1_ref[...], preferred_element_type=jnp.float32) + bm1_ref[...]
    m = jax.nn.sigmoid(_bn(z, gm1_ref[...], bem1_ref[...]))
    hp_ref[...] = (jnp.dot(m, wm2_ref[...], preferred_element_type=jnp.float32)
                   + bm2_ref[...])


def _dense_b(x_ref, aggx_ref, aggan_ref, deg_ref, w1s_ref, w1n_ref, b1_ref,
             g1_ref, be1_ref, h_ref, aggan1_ref):
    inv = 1.0 / jnp.maximum(deg_ref[...], 1.0)
    mean1 = (aggx_ref[0, :N] + aggx_ref[1, :N]) * inv
    aggan1_ref[...] = aggan_ref[0, :N] + aggan_ref[1, :N]
    z = (jnp.dot(x_ref[...], w1s_ref[...], preferred_element_type=jnp.float32)
         + jnp.dot(mean1, w1n_ref[...], preferred_element_type=jnp.float32)
         + b1_ref[...])
    h_ref[...] = jnp.maximum(_bn(z, g1_ref[...], be1_ref[...]), 0.0)


def _dense_c(h_ref, aggh_ref, deg_ref, an_ref, aggan_ref, ss_ref, hp_ref,
             w2s_ref, w2n_ref, b2_ref, temp_ref, wc_ref, bc_ref, out_ref):
    deg = deg_ref[...]
    inv = 1.0 / jnp.maximum(deg, 1.0)
    mean2 = (aggh_ref[0, :N] + aggh_ref[1, :N]) * inv
    hg = (jnp.dot(h_ref[...], w2s_ref[...], preferred_element_type=jnp.float32)
          + jnp.dot(mean2, w2n_ref[...], preferred_element_type=jnp.float32)
          + b2_ref[...])
    ssum = jnp.sum(an_ref[...] * aggan_ref[...], axis=1, keepdims=True) + ss_ref[...]
    gate = jax.nn.sigmoid(temp_ref[...] * ssum / (deg + 1.0))
    z = jnp.maximum(gate * hg + (1.0 - gate) * hp_ref[...], 0.0)
    logits = (jnp.dot(z, wc_ref[...], preferred_element_type=jnp.float32)
              + bc_ref[...])
    mx = jnp.max(logits, axis=1, keepdims=True)
    e = jnp.exp(logits - mx)
    out_ref[...] = e / jnp.sum(e, axis=1, keepdims=True)


def kernel(x, alpha, edge_index, W1s, W1n, b1, g1, be1, W2s, W2n, b2,
           Wm1, bm1, gm1, bem1, Wm2, bm2, temp, Wc, bc):
    f32 = jnp.float32
    src = edge_index[0].reshape(NW, CH_PW, K)
    tgt = edge_index[1].reshape(NW, CH_PW, K)
    ones_k = jnp.ones((K,), f32)
    zrow = jnp.zeros((NR, D), f32)
    zdeg = jnp.zeros((NR,), f32)

    an, hp, ss = pl.pallas_call(
        _dense_a,
        out_shape=(jax.ShapeDtypeStruct((N, D), f32),
                   jax.ShapeDtypeStruct((N, D), f32),
                   jax.ShapeDtypeStruct((N, 1), f32)),
    )(alpha, Wm1, bm1.reshape(1, -1), gm1.reshape(1, -1),
      bem1.reshape(1, -1), Wm2, bm2.reshape(1, -1))

    aggx, degp = _seg_deg(x, src, tgt, ones_k, zrow, zdeg)
    aggan = _seg(an, src, tgt, zrow)
    deg = (degp[0, :N] + degp[1, :N]).reshape(N, 1)

    h, aggan1 = pl.pallas_call(
        _dense_b,
        out_shape=(jax.ShapeDtypeStruct((N, D), f32),
                   jax.ShapeDtypeStruct((N, D), f32)),
    )(x, aggx, aggan, deg, W1s, W1n, b1.reshape(1, -1), g1.reshape(1, -1),
      be1.reshape(1, -1))

    aggh = _seg(h, src, tgt, zrow)

    out = pl.pallas_call(
        _dense_c,
        out_shape=jax.ShapeDtypeStruct((N, 40), f32),
    )(h, aggh, deg, an, aggan1, ss, hp, W2s, W2n, b2.reshape(1, -1),
      temp.reshape(1, 1), Wc, bc.reshape(1, -1))
    return out


# speedup vs baseline: 6.9160x; 1.0523x over previous
import functools

import jax
import jax.numpy as jnp
from jax import lax
from jax.experimental import pallas as pl
from jax.experimental.pallas import tpu as pltpu
from jax.experimental.pallas import tpu_sc as plsc

N = 10000
D = 128
E = 320000
K = 50
NC = 2
NS = 16
NW = NC * NS
NR = 10240
ROWS_PT = NR // NS
CHUNKS = E // K
CH_PW = CHUNKS // NW
IB = 25
NB = CH_PW // IB

_MESH = plsc.VectorSubcoreMesh(core_axis_name="c", subcore_axis_name="s")


def _seg_body(with_deg, table_hbm, srcr_hbm, tgtr_hbm, ones_hbm, zrow_hbm,
              zdeg_hbm, agg_hbm, deg_hbm, sidx, tidx, rows, onev, acc, dacc,
              gsem0, gsem1, isem, dsem):
    c = lax.axis_index("c")
    s = lax.axis_index("s")
    w = s * NC + c
    pltpu.sync_copy(zrow_hbm.at[pl.ds(s * ROWS_PT, ROWS_PT)],
                    acc.at[pl.ds(s * ROWS_PT, ROWS_PT)])
    if with_deg:
        pltpu.sync_copy(ones_hbm, onev)
        pltpu.sync_copy(zdeg_hbm.at[pl.ds(s * ROWS_PT, ROWS_PT)],
                        dacc.at[pl.ds(s * ROWS_PT, ROWS_PT)])

    plsc.subcore_barrier()
    my_src = srcr_hbm.at[w]
    my_tgt = tgtr_hbm.at[w]

    def gather(blk, jj, buf, sem):
        pltpu.async_copy(table_hbm.at[sidx.at[blk, jj]], rows.at[buf], sem)

    def gwait(blk, jj, buf, sem):
        pltpu.make_async_copy(table_hbm.at[sidx.at[blk, jj]], rows.at[buf],
                              sem).wait()

    pltpu.sync_copy(my_src.at[0], sidx.at[0])
    pltpu.sync_copy(my_tgt.at[0], tidx.at[0])

    def outer(b, carry):
        hb = lax.rem(b, 2)
        @pl.when(b > 0)
        def _():
            pltpu.make_async_copy(my_src.at[b], sidx.at[hb], isem).wait()
            pltpu.make_async_copy(my_tgt.at[b], tidx.at[hb], isem).wait()

        @pl.when(b + 1 < NB)
        def _():
            nh = lax.rem(b + 1, 2)
            pltpu.async_copy(my_src.at[b + 1], sidx.at[nh], isem)
            pltpu.async_copy(my_tgt.at[b + 1], tidx.at[nh], isem)

        p0 = lax.rem(b * IB, 2)

        @pl.when(p0 == 0)
        def _():
            gather(hb, 0, 0, gsem0)

        @pl.when(p0 == 1)
        def _():
            gather(hb, 0, 1, gsem1)

        def inner(jj, carry2):
            j = b * IB + jj
            p = lax.rem(j, 2)

            @pl.when(p == 0)
            def _():
                gwait(hb, jj, 0, gsem0)

            @pl.when(p == 1)
            def _():
                gwait(hb, jj, 1, gsem1)

            @pl.when(jj + 1 < IB)
            def _():
                @pl.when(p == 0)
                def _():
                    gather(hb, jj + 1, 1, gsem1)

                @pl.when(p == 1)
                def _():
                    gather(hb, jj + 1, 0, gsem0)

            if with_deg:
                @pl.when(j > 0)
                def _():
                    pltpu.make_async_copy(onev, dacc.at[tidx.at[hb, jj]],
                                          dsem).wait()
                pltpu.async_copy(onev, dacc.at[tidx.at[hb, jj]], dsem,
                                 add=True)
            pltpu.sync_copy(rows.at[p], acc.at[tidx.at[hb, jj]], add=True)
            return carry2

        lax.fori_loop(0, IB, inner, 0)
        return carry

    lax.fori_loop(0, NB, outer, 0)
    if with_deg:
        pltpu.make_async_copy(onev, dacc.at[tidx.at[lax.rem(NB - 1, 2),
                                                    IB - 1]], dsem).wait()
    plsc.subcore_barrier()
    pltpu.sync_copy(acc.at[pl.ds(s * ROWS_PT, ROWS_PT)],
                    agg_hbm.at[c].at[pl.ds(s * ROWS_PT, ROWS_PT)])
    if with_deg:
        pltpu.sync_copy(dacc.at[pl.ds(s * ROWS_PT, ROWS_PT)],
                        deg_hbm.at[c].at[pl.ds(s * ROWS_PT, ROWS_PT)])


_seg_deg = functools.partial(
    pl.kernel,
    out_type=(jax.ShapeDtypeStruct((NC, NR, D), jnp.float32),
              jax.ShapeDtypeStruct((NC, NR), jnp.float32)),
    mesh=_MESH,
    scratch_types=[
        pltpu.VMEM((2, IB, K), jnp.int32),
        pltpu.VMEM((2, IB, K), jnp.int32),
        pltpu.VMEM((2, K, D), jnp.float32),
        pltpu.VMEM((K,), jnp.float32),
        pltpu.VMEM_SHARED((NR, D), jnp.float32),
        pltpu.VMEM_SHARED((NR,), jnp.float32),
        pltpu.SemaphoreType.DMA,
        pltpu.SemaphoreType.DMA,
        pltpu.SemaphoreType.DMA,
        pltpu.SemaphoreType.DMA,
    ],
)(functools.partial(_seg_body, True))


def _seg_nodeg_body(table_hbm, srcr_hbm, tgtr_hbm, zrow_hbm, agg_hbm,
                    sidx, tidx, rows, acc, gsem0, gsem1, isem):
    _seg_body(False, table_hbm, srcr_hbm, tgtr_hbm, None, zrow_hbm, None,
              agg_hbm, None, sidx, tidx, rows, None, acc, None,
              gsem0, gsem1, isem, None)


_seg = functools.partial(
    pl.kernel,
    out_type=jax.ShapeDtypeStruct((NC, NR, D), jnp.float32),
    mesh=_MESH,
    scratch_types=[
        pltpu.VMEM((2, IB, K), jnp.int32),
        pltpu.VMEM((2, IB, K), jnp.int32),
        pltpu.VMEM((2, K, D), jnp.float32),
        pltpu.VMEM_SHARED((NR, D), jnp.float32),
        pltpu.SemaphoreType.DMA,
        pltpu.SemaphoreType.DMA,
        pltpu.SemaphoreType.DMA,
    ],
)(_seg_nodeg_body)


def _bn(z, g, be):
    mu = jnp.mean(z, axis=0, keepdims=True)
    zc = z - mu
    var = jnp.mean(zc * zc, axis=0, keepdims=True)
    return g * zc * lax.rsqrt(var + 1e-5) + be


def _dense_a(al_ref, wm1_ref, bm1_ref, gm1_ref, bem1_ref, wm2_ref,
             bm2_ref, an_ref, hp_ref, ss_ref):
    al = al_ref[...]
    nrm = jnp.sqrt(jnp.sum(al * al, axis=1, keepdims=True))
    an = al / jnp.maximum(nrm, 1e-12)
    an_ref[...] = an
    ss_ref[...] = jnp.sum(an * an, axis=1, keepdims=True)
    z = jnp.dot(al, wm1_ref[...], preferred_element_type=jnp.float32) + bm1_ref[...]
    m = jax.nn.sigmoid(_bn(z, gm1_ref[...], bem1_ref[...]))
    hp_ref[...] = (jnp.dot(m, wm2_ref[...], preferred_element_type=jnp.float32)
                   + bm2_ref[...])


def _dense_b(x_ref, aggx_ref, aggan_ref, deg_ref, w1s_ref, w1n_ref, b1_ref,
             g1_ref, be1_ref, h_ref, aggan1_ref):
    inv = 1.0 / jnp.maximum(deg_ref[...], 1.0)
    mean1 = (aggx_ref[0, :N] + aggx_ref[1, :N]) * inv
    aggan1_ref[...] = aggan_ref[0, :N] + aggan_ref[1, :N]
    z = (jnp.dot(x_ref[...], w1s_ref[...], preferred_element_type=jnp.float32)
         + jnp.dot(mean1, w1n_ref[...], preferred_element_type=jnp.float32)
         + b1_ref[...])
    h_ref[...] = jnp.maximum(_bn(z, g1_ref[...], be1_ref[...]), 0.0)


def _dense_c(h_ref, aggh_ref, deg_ref, an_ref, aggan_ref, ss_ref, hp_ref,
             w2s_ref, w2n_ref, b2_ref, temp_ref, wc_ref, bc_ref, out_ref):
    deg = deg_ref[...]
    inv = 1.0 / jnp.maximum(deg, 1.0)
    mean2 = (aggh_ref[0, :N] + aggh_ref[1, :N]) * inv
    hg = (jnp.dot(h_ref[...], w2s_ref[...], preferred_element_type=jnp.float32)
          + jnp.dot(mean2, w2n_ref[...], preferred_element_type=jnp.float32)
          + b2_ref[...])
    ssum = jnp.sum(an_ref[...] * aggan_ref[...], axis=1, keepdims=True) + ss_ref[...]
    gate = jax.nn.sigmoid(temp_ref[...] * ssum / (deg + 1.0))
    z = jnp.maximum(gate * hg + (1.0 - gate) * hp_ref[...], 0.0)
    logits = (jnp.dot(z, wc_ref[...], preferred_element_type=jnp.float32)
              + bc_ref[...])
    mx = jnp.max(logits, axis=1, keepdims=True)
    e = jnp.exp(logits - mx)
    out_ref[...] = e / jnp.sum(e, axis=1, keepdims=True)


def kernel(x, alpha, edge_index, W1s, W1n, b1, g1, be1, W2s, W2n, b2,
           Wm1, bm1, gm1, bem1, Wm2, bm2, temp, Wc, bc):
    f32 = jnp.float32
    src = edge_index[0].reshape(NW, NB, IB, K)
    tgt = edge_index[1].reshape(NW, NB, IB, K)
    ones_k = jnp.ones((K,), f32)
    zrow = jnp.zeros((NR, D), f32)
    zdeg = jnp.zeros((NR,), f32)

    an, hp, ss = pl.pallas_call(
        _dense_a,
        out_shape=(jax.ShapeDtypeStruct((N, D), f32),
                   jax.ShapeDtypeStruct((N, D), f32),
                   jax.ShapeDtypeStruct((N, 1), f32)),
    )(alpha, Wm1, bm1.reshape(1, -1), gm1.reshape(1, -1),
      bem1.reshape(1, -1), Wm2, bm2.reshape(1, -1))

    aggx, degp = _seg_deg(x, src, tgt, ones_k, zrow, zdeg)
    aggan = _seg(an, src, tgt, zrow)
    deg = (degp[0, :N] + degp[1, :N]).reshape(N, 1)

    h, aggan1 = pl.pallas_call(
        _dense_b,
        out_shape=(jax.ShapeDtypeStruct((N, D), f32),
                   jax.ShapeDtypeStruct((N, D), f32)),
    )(x, aggx, aggan, deg, W1s, W1n, b1.reshape(1, -1), g1.reshape(1, -1),
      be1.reshape(1, -1))

    aggh = _seg(h, src, tgt, zrow)

    out = pl.pallas_call(
        _dense_c,
        out_shape=jax.ShapeDtypeStruct((N, 40), f32),
    )(h, aggh, deg, an, aggan1, ss, hp, W2s, W2n, b2.reshape(1, -1),
      temp.reshape(1, 1), Wc, bc.reshape(1, -1))
    return out


# speedup vs baseline: 6.9520x; 1.0052x over previous
import functools

import jax
import jax.numpy as jnp
from jax import lax
from jax.experimental import pallas as pl
from jax.experimental.pallas import tpu as pltpu
from jax.experimental.pallas import tpu_sc as plsc

N = 10000
D = 128
E = 320000
K = 50
NC = 2
NS = 16
NW = NC * NS
NR = 10240
ROWS_PT = NR // NS
CHUNKS = E // K
CH_PW = CHUNKS // NW
IB = 25
NB = CH_PW // IB

_MESH = plsc.VectorSubcoreMesh(core_axis_name="c", subcore_axis_name="s")


def _seg_body(with_deg, table_hbm, srcr_hbm, tgtr_hbm, ones_hbm, zrow_hbm,
              zdeg_hbm, agg_hbm, deg_hbm, sidx, tidx, rows, onev, acc, dacc,
              gsem0, gsem1, isem, dsem, ssem0, ssem1):
    c = lax.axis_index("c")
    s = lax.axis_index("s")
    w = s * NC + c
    pltpu.sync_copy(zrow_hbm.at[pl.ds(s * ROWS_PT, ROWS_PT)],
                    acc.at[pl.ds(s * ROWS_PT, ROWS_PT)])
    if with_deg:
        pltpu.sync_copy(ones_hbm, onev)
        pltpu.sync_copy(zdeg_hbm.at[pl.ds(s * ROWS_PT, ROWS_PT)],
                        dacc.at[pl.ds(s * ROWS_PT, ROWS_PT)])

    plsc.subcore_barrier()
    my_src = srcr_hbm.at[w]
    my_tgt = tgtr_hbm.at[w]

    def gather(blk, jj, buf, sem):
        pltpu.async_copy(table_hbm.at[sidx.at[blk, jj]], rows.at[buf], sem)

    def gwait(blk, jj, buf, sem):
        pltpu.make_async_copy(table_hbm.at[sidx.at[blk, jj]], rows.at[buf],
                              sem).wait()

    def swait(blk, jj, buf, sem):
        pltpu.make_async_copy(rows.at[buf], acc.at[tidx.at[blk, jj]],
                              sem).wait()

    pltpu.sync_copy(my_src.at[0], sidx.at[0])
    pltpu.sync_copy(my_tgt.at[0], tidx.at[0])

    def outer(b, carry):
        hb = lax.rem(b, 2)
        @pl.when(b > 0)
        def _():
            pltpu.make_async_copy(my_src.at[b], sidx.at[hb], isem).wait()
            pltpu.make_async_copy(my_tgt.at[b], tidx.at[hb], isem).wait()

        @pl.when(b + 1 < NB)
        def _():
            nh = lax.rem(b + 1, 2)
            pltpu.async_copy(my_src.at[b + 1], sidx.at[nh], isem)
            pltpu.async_copy(my_tgt.at[b + 1], tidx.at[nh], isem)

        p0 = lax.rem(b * IB, 2)

        @pl.when(p0 == 0)
        def _():
            @pl.when(b * IB >= 2)
            def _():
                swait(hb, 0, 0, ssem0)
            gather(hb, 0, 0, gsem0)

        @pl.when(p0 == 1)
        def _():
            @pl.when(b * IB >= 2)
            def _():
                swait(hb, 0, 1, ssem1)
            gather(hb, 0, 1, gsem1)

        def inner(jj, carry2):
            j = b * IB + jj
            p = lax.rem(j, 2)

            @pl.when(p == 0)
            def _():
                gwait(hb, jj, 0, gsem0)

            @pl.when(p == 1)
            def _():
                gwait(hb, jj, 1, gsem1)

            @pl.when(jj + 1 < IB)
            def _():
                @pl.when(p == 0)
                def _():
                    @pl.when(j >= 1)
                    def _():
                        swait(hb, jj + 1, 1, ssem1)
                    gather(hb, jj + 1, 1, gsem1)

                @pl.when(p == 1)
                def _():
                    @pl.when(j >= 1)
                    def _():
                        swait(hb, jj + 1, 0, ssem0)
                    gather(hb, jj + 1, 0, gsem0)

            if with_deg:
                @pl.when(j > 0)
                def _():
                    pltpu.make_async_copy(onev, dacc.at[tidx.at[hb, jj]],
                                          dsem).wait()
                pltpu.async_copy(onev, dacc.at[tidx.at[hb, jj]], dsem,
                                 add=True)
            @pl.when(p == 0)
            def _():
                pltpu.async_copy(rows.at[0], acc.at[tidx.at[hb, jj]], ssem0,
                                 add=True)

            @pl.when(p == 1)
            def _():
                pltpu.async_copy(rows.at[1], acc.at[tidx.at[hb, jj]], ssem1,
                                 add=True)
            return carry2

        lax.fori_loop(0, IB, inner, 0)
        return carry

    lax.fori_loop(0, NB, outer, 0)
    lastb = lax.rem(NB - 1, 2)
    lastp = lax.rem(CH_PW - 1, 2)

    @pl.when(lastp == 0)
    def _():
        swait(lastb, IB - 1, 0, ssem0)
        swait(lastb, IB - 2, 1, ssem1)

    @pl.when(lastp == 1)
    def _():
        swait(lastb, IB - 1, 1, ssem1)
        swait(lastb, IB - 2, 0, ssem0)

    if with_deg:
        pltpu.make_async_copy(onev, dacc.at[tidx.at[lax.rem(NB - 1, 2),
                                                    IB - 1]], dsem).wait()
    plsc.subcore_barrier()
    pltpu.sync_copy(acc.at[pl.ds(s * ROWS_PT, ROWS_PT)],
                    agg_hbm.at[c].at[pl.ds(s * ROWS_PT, ROWS_PT)])
    if with_deg:
        pltpu.sync_copy(dacc.at[pl.ds(s * ROWS_PT, ROWS_PT)],
                        deg_hbm.at[c].at[pl.ds(s * ROWS_PT, ROWS_PT)])


_seg_deg = functools.partial(
    pl.kernel,
    out_type=(jax.ShapeDtypeStruct((NC, NR, D), jnp.float32),
              jax.ShapeDtypeStruct((NC, NR), jnp.float32)),
    mesh=_MESH,
    scratch_types=[
        pltpu.VMEM((2, IB, K), jnp.int32),
        pltpu.VMEM((2, IB, K), jnp.int32),
        pltpu.VMEM((2, K, D), jnp.float32),
        pltpu.VMEM((K,), jnp.float32),
        pltpu.VMEM_SHARED((NR, D), jnp.float32),
        pltpu.VMEM_SHARED((NR,), jnp.float32),
        pltpu.SemaphoreType.DMA,
        pltpu.SemaphoreType.DMA,
        pltpu.SemaphoreType.DMA,
        pltpu.SemaphoreType.DMA,
        pltpu.SemaphoreType.DMA,
        pltpu.SemaphoreType.DMA,
    ],
)(functools.partial(_seg_body, True))


def _seg_nodeg_body(table_hbm, srcr_hbm, tgtr_hbm, zrow_hbm, agg_hbm,
                    sidx, tidx, rows, acc, gsem0, gsem1, isem, ssem0, ssem1):
    _seg_body(False, table_hbm, srcr_hbm, tgtr_hbm, None, zrow_hbm, None,
              agg_hbm, None, sidx, tidx, rows, None, acc, None,
              gsem0, gsem1, isem, None, ssem0, ssem1)


_seg = functools.partial(
    pl.kernel,
    out_type=jax.ShapeDtypeStruct((NC, NR, D), jnp.float32),
    mesh=_MESH,
    scratch_types=[
        pltpu.VMEM((2, IB, K), jnp.int32),
        pltpu.VMEM((2, IB, K), jnp.int32),
        pltpu.VMEM((2, K, D), jnp.float32),
        pltpu.VMEM_SHARED((NR, D), jnp.float32),
        pltpu.SemaphoreType.DMA,
        pltpu.SemaphoreType.DMA,
        pltpu.SemaphoreType.DMA,
        pltpu.SemaphoreType.DMA,
        pltpu.SemaphoreType.DMA,
    ],
)(_seg_nodeg_body)


def _bn(z, g, be):
    mu = jnp.mean(z, axis=0, keepdims=True)
    zc = z - mu
    var = jnp.mean(zc * zc, axis=0, keepdims=True)
    return g * zc * lax.rsqrt(var + 1e-5) + be


def _dense_a(al_ref, wm1_ref, bm1_ref, gm1_ref, bem1_ref, wm2_ref,
             bm2_ref, an_ref, hp_ref, ss_ref):
    al = al_ref[...]
    nrm = jnp.sqrt(jnp.sum(al * al, axis=1, keepdims=True))
    an = al / jnp.maximum(nrm, 1e-12)
    an_ref[...] = an
    ss_ref[...] = jnp.sum(an * an, axis=1, keepdims=True)
    z = jnp.dot(al, wm1_ref[...], preferred_element_type=jnp.float32) + bm1_ref[...]
    m = jax.nn.sigmoid(_bn(z, gm1_ref[...], bem1_ref[...]))
    hp_ref[...] = (jnp.dot(m, wm2_ref[...], preferred_element_type=jnp.float32)
                   + bm2_ref[...])


def _dense_b(x_ref, aggx_ref, aggan_ref, deg_ref, w1s_ref, w1n_ref, b1_ref,
             g1_ref, be1_ref, h_ref, aggan1_ref):
    inv = 1.0 / jnp.maximum(deg_ref[...], 1.0)
    mean1 = (aggx_ref[0, :N] + aggx_ref[1, :N]) * inv
    aggan1_ref[...] = aggan_ref[0, :N] + aggan_ref[1, :N]
    z = (jnp.dot(x_ref[...], w1s_ref[...], preferred_element_type=jnp.float32)
         + jnp.dot(mean1, w1n_ref[...], preferred_element_type=jnp.float32)
         + b1_ref[...])
    h_ref[...] = jnp.maximum(_bn(z, g1_ref[...], be1_ref[...]), 0.0)


def _dense_c(h_ref, aggh_ref, deg_ref, an_ref, aggan_ref, ss_ref, hp_ref,
             w2s_ref, w2n_ref, b2_ref, temp_ref, wc_ref, bc_ref, out_ref):
    deg = deg_ref[...]
    inv = 1.0 / jnp.maximum(deg, 1.0)
    mean2 = (aggh_ref[0, :N] + aggh_ref[1, :N]) * inv
    hg = (jnp.dot(h_ref[...], w2s_ref[...], preferred_element_type=jnp.float32)
          + jnp.dot(mean2, w2n_ref[...], preferred_element_type=jnp.float32)
          + b2_ref[...])
    ssum = jnp.sum(an_ref[...] * aggan_ref[...], axis=1, keepdims=True) + ss_ref[...]
    gate = jax.nn.sigmoid(temp_ref[...] * ssum / (deg + 1.0))
    z = jnp.maximum(gate * hg + (1.0 - gate) * hp_ref[...], 0.0)
    logits = (jnp.dot(z, wc_ref[...], preferred_element_type=jnp.float32)
              + bc_ref[...])
    mx = jnp.max(logits, axis=1, keepdims=True)
    e = jnp.exp(logits - mx)
    out_ref[...] = e / jnp.sum(e, axis=1, keepdims=True)


def kernel(x, alpha, edge_index, W1s, W1n, b1, g1, be1, W2s, W2n, b2,
           Wm1, bm1, gm1, bem1, Wm2, bm2, temp, Wc, bc):
    f32 = jnp.float32
    src = edge_index[0].reshape(NW, NB, IB, K)
    tgt = edge_index[1].reshape(NW, NB, IB, K)
    ones_k = jnp.ones((K,), f32)
    zrow = jnp.zeros((NR, D), f32)
    zdeg = jnp.zeros((NR,), f32)

    an, hp, ss = pl.pallas_call(
        _dense_a,
        out_shape=(jax.ShapeDtypeStruct((N, D), f32),
                   jax.ShapeDtypeStruct((N, D), f32),
                   jax.ShapeDtypeStruct((N, 1), f32)),
    )(alpha, Wm1, bm1.reshape(1, -1), gm1.reshape(1, -1),
      bem1.reshape(1, -1), Wm2, bm2.reshape(1, -1))

    aggx, degp = _seg_deg(x, src, tgt, ones_k, zrow, zdeg)
    aggan = _seg(an, src, tgt, zrow)
    deg = (degp[0, :N] + degp[1, :N]).reshape(N, 1)

    h, aggan1 = pl.pallas_call(
        _dense_b,
        out_shape=(jax.ShapeDtypeStruct((N, D), f32),
                   jax.ShapeDtypeStruct((N, D), f32)),
    )(x, aggx, aggan, deg, W1s, W1n, b1.reshape(1, -1), g1.reshape(1, -1),
      be1.reshape(1, -1))

    aggh = _seg(h, src, tgt, zrow)

    out = pl.pallas_call(
        _dense_c,
        out_shape=jax.ShapeDtypeStruct((N, 40), f32),
    )(h, aggh, deg, an, aggan1, ss, hp, W2s, W2n, b2.reshape(1, -1),
      temp.reshape(1, 1), Wc, bc.reshape(1, -1))
    return out
